# baseline (speedup 1.0000x reference)
import functools

import jax
import jax.numpy as jnp
from jax import lax
from jax.experimental import pallas as pl
from jax.experimental.pallas import tpu as pltpu
from jax.experimental.pallas import tpu_sc as plsc

BSZ, TSZ, FSZ = 8, 256, 512
NUM_GROUPS, NUM_VARS, VAR_DIM = 2, 320, 128
NUM_TOTAL = NUM_GROUPS * NUM_VARS
TT = 128
NSTEPS = TSZ // TT
ROWS = BSZ * TT

_B_TOTAL = BSZ * TSZ * NUM_GROUPS


def _vq_tc_kernel(x_ref, w_ref, b_ref, idx_ref, cb_ref, cpp_ref, ppp_ref,
                  acc_ref):
    i = pl.program_id(0)

    @pl.when(i == 0)
    def _init():
        acc_ref[...] = jnp.zeros_like(acc_ref)
        cpp_ref[...] = jnp.zeros_like(cpp_ref)
        ppp_ref[...] = jnp.zeros_like(ppp_ref)

    xb = x_ref[...].reshape(ROWS, FSZ)
    logits = jax.lax.dot_general(
        xb, w_ref[...], (((1,), (1,)), ((), ())),
        preferred_element_type=jnp.float32)
    logits = logits + b_ref[...]

    col = jax.lax.broadcasted_iota(jnp.int32, (ROWS, NUM_TOTAL), 1)
    g0 = col < NUM_VARS
    neg = jnp.float32(-jnp.inf)

    l0 = jnp.where(g0, logits, neg)
    l1 = jnp.where(g0, neg, logits)
    m0 = jnp.max(l0, axis=1, keepdims=True)
    m1 = jnp.max(l1, axis=1, keepdims=True)
    a0 = jnp.argmax(l0, axis=1).reshape(ROWS, 1)
    a1 = jnp.argmax(l1, axis=1).reshape(ROWS, 1)
    oh0 = (col == a0).astype(jnp.float32)
    oh1 = (col == a1).astype(jnp.float32)
    oh = oh0 + oh1

    cb_ref[...] = oh.reshape(BSZ, TT, NUM_TOTAL)
    idx_ref[...] = jnp.concatenate([a0, a1], axis=1).reshape(BSZ, TT, 2)

    counts = oh.reshape(BSZ, TT, NUM_TOTAL).sum(axis=0) * (1.0 / BSZ)
    colt = jax.lax.broadcasted_iota(jnp.int32, (TT, NUM_TOTAL), 1)
    ent = counts * jnp.log(counts + 1e-07)
    s0 = jnp.sum(jnp.where(colt < NUM_VARS, ent, 0.0), axis=1, keepdims=True)
    s1 = jnp.sum(jnp.where(colt < NUM_VARS, 0.0, ent), axis=1, keepdims=True)
    cpp_ref[...] += jnp.sum(jnp.exp(-s0) + jnp.exp(-s1)).reshape(1, 1)

    m_sel = jnp.where(g0, m0, m1)
    e = jnp.exp(logits - m_sel)
    gcol = jax.lax.broadcasted_iota(jnp.int32, (NUM_TOTAL, 2), 0)
    gidx = jax.lax.broadcasted_iota(jnp.int32, (NUM_TOTAL, 2), 1)
    gmask = ((gcol < NUM_VARS) == (gidx == 0)).astype(jnp.float32)
    se2 = jax.lax.dot_general(e, gmask, (((1,), (0,)), ((), ())),
                              preferred_element_type=jnp.float32)
    w2 = 1.0 / se2
    accp = jax.lax.dot_general(e, w2, (((0,), (0,)), ((), ())),
                               preferred_element_type=jnp.float32)
    acc_ref[...] += accp

    @pl.when(i == NSTEPS - 1)
    def _finalize():
        pavg = acc_ref[...] * (1.0 / (BSZ * TSZ))
        entp = pavg * jnp.log(pavg + 1e-07) * gmask
        sp = jnp.sum(entp, axis=0, keepdims=True)
        ppp_ref[...] = jnp.sum(jnp.exp(-sp)).reshape(1, 1)


@functools.lru_cache(maxsize=1)
def _make_sc_gather():
    info = plsc.get_sparse_core_info()
    nc, ns = info.num_cores, info.num_subcores
    b_per_w = _B_TOTAL // (nc * ns)

    def _sc_gather_kernel(table_hbm, idx_hbm, out_hbm, idx_v, rows_v, sem):
        wid = lax.axis_index("s") * nc + lax.axis_index("c")
        base = wid * b_per_w
        pltpu.sync_copy(idx_hbm.at[pl.ds(base, b_per_w)], idx_v)
        pltpu.async_copy(table_hbm.at[idx_v], rows_v, sem).wait()
        pltpu.sync_copy(rows_v, out_hbm.at[pl.ds(base, b_per_w)])

    return pl.kernel(
        _sc_gather_kernel,
        mesh=plsc.VectorSubcoreMesh(core_axis_name="c", subcore_axis_name="s"),
        out_type=jax.ShapeDtypeStruct((_B_TOTAL, VAR_DIM), jnp.float32),
        scratch_types=[
            pltpu.VMEM((b_per_w,), jnp.int32),
            pltpu.VMEM((b_per_w, VAR_DIM), jnp.float32),
            pltpu.SemaphoreType.DMA,
        ],
    )


@jax.jit
def _run(x, W, b, vars_):
    b2 = b.reshape(1, NUM_TOTAL)

    idx, cb, cpp, ppp = pl.pallas_call(
        _vq_tc_kernel,
        grid=(NSTEPS,),
        in_specs=[
            pl.BlockSpec((BSZ, TT, FSZ), lambda i: (0, i, 0)),
            pl.BlockSpec((NUM_TOTAL, FSZ), lambda i: (0, 0)),
            pl.BlockSpec((1, NUM_TOTAL), lambda i: (0, 0)),
        ],
        out_specs=[
            pl.BlockSpec((BSZ, TT, 2), lambda i: (0, i, 0)),
            pl.BlockSpec((BSZ, TT, NUM_TOTAL), lambda i: (0, i, 0)),
            pl.BlockSpec((1, 1), lambda i: (0, 0)),
            pl.BlockSpec((1, 1), lambda i: (0, 0)),
        ],
        out_shape=[
            jax.ShapeDtypeStruct((BSZ, TSZ, 2), jnp.int32),
            jax.ShapeDtypeStruct((BSZ, TSZ, NUM_TOTAL), jnp.float32),
            jax.ShapeDtypeStruct((1, 1), jnp.float32),
            jax.ShapeDtypeStruct((1, 1), jnp.float32),
        ],
        scratch_shapes=[pltpu.VMEM((NUM_TOTAL, 2), jnp.float32)],
    )(x, W, b2)

    qrows = _make_sc_gather()(vars_.reshape(NUM_TOTAL, VAR_DIM),
                              idx.reshape(_B_TOTAL))
    q = qrows.reshape(BSZ, TSZ, NUM_GROUPS * VAR_DIM)
    return q, cb.reshape(BSZ * TSZ, NUM_TOTAL), cpp[0, 0], ppp[0, 0]


def kernel(x, W, b, vars_):
    return _run(x, W, b, vars_)


# speedup vs baseline: 2.9729x; 2.9729x over previous
import functools

import jax
import jax.numpy as jnp
from jax.experimental import pallas as pl
from jax.experimental.pallas import tpu as pltpu

BSZ, TSZ, FSZ = 8, 256, 512
NUM_GROUPS, NUM_VARS, VAR_DIM = 2, 320, 128
NUM_TOTAL = NUM_GROUPS * NUM_VARS
TT = 128
NSTEPS = TSZ // TT
ROWS = BSZ * TT


def _vq_kernel(x_ref, w_ref, b_ref, v_ref, q_ref, cb_ref, cpp_ref, ppp_ref,
               acc_ref):
    i = pl.program_id(0)

    @pl.when(i == 0)
    def _init():
        acc_ref[...] = jnp.zeros_like(acc_ref)
        cpp_ref[...] = jnp.zeros_like(cpp_ref)
        ppp_ref[...] = jnp.zeros_like(ppp_ref)

    xb = x_ref[...].reshape(ROWS, FSZ)
    logits = jax.lax.dot_general(
        xb, w_ref[...], (((1,), (1,)), ((), ())),
        preferred_element_type=jnp.float32)
    logits = logits + b_ref[...]

    col = jax.lax.broadcasted_iota(jnp.int32, (ROWS, NUM_TOTAL), 1)
    g0 = col < NUM_VARS
    neg = jnp.float32(-jnp.inf)

    m0 = jnp.max(jnp.where(g0, logits, neg), axis=1, keepdims=True)
    m1 = jnp.max(jnp.where(g0, neg, logits), axis=1, keepdims=True)

    big = jnp.int32(NUM_TOTAL)
    a0 = jnp.min(jnp.where(g0 & (logits == m0), col, big), axis=1, keepdims=True)
    a1 = jnp.min(jnp.where((~g0) & (logits == m1), col, big), axis=1, keepdims=True)
    oh0 = (col == a0).astype(jnp.float32)
    oh1 = (col == a1).astype(jnp.float32)
    oh = oh0 + oh1

    cb_ref[...] = oh.reshape(BSZ, TT, NUM_TOTAL)

    v = v_ref[0]
    q = jnp.concatenate(
        [jnp.dot(oh0, v, preferred_element_type=jnp.float32),
         jnp.dot(oh1, v, preferred_element_type=jnp.float32)], axis=1)
    q_ref[...] = q.reshape(BSZ, TT, NUM_GROUPS * VAR_DIM)

    counts = oh.reshape(BSZ, TT, NUM_TOTAL).sum(axis=0) * (1.0 / BSZ)
    colt = jax.lax.broadcasted_iota(jnp.int32, (TT, NUM_TOTAL), 1)
    ent = counts * jnp.log(counts + 1e-07)
    s0 = jnp.sum(jnp.where(colt < NUM_VARS, ent, 0.0), axis=1, keepdims=True)
    s1 = jnp.sum(jnp.where(colt < NUM_VARS, 0.0, ent), axis=1, keepdims=True)
    cpp_ref[...] += jnp.sum(jnp.exp(-s0) + jnp.exp(-s1)).reshape(1, 1)

    m_sel = jnp.where(g0, m0, m1)
    e = jnp.exp(logits - m_sel)
    se0 = jnp.sum(jnp.where(g0, e, 0.0), axis=1, keepdims=True)
    se1 = jnp.sum(jnp.where(g0, 0.0, e), axis=1, keepdims=True)
    p = e / jnp.where(g0, se0, se1)
    acc_ref[0:1, :] += jnp.sum(p, axis=0, keepdims=True)

    @pl.when(i == NSTEPS - 1)
    def _finalize():
        pavg = acc_ref[0:1, :] * (1.0 / (BSZ * TSZ))
        entp = pavg * jnp.log(pavg + 1e-07)
        colp = jax.lax.broadcasted_iota(jnp.int32, (1, NUM_TOTAL), 1)
        sp0 = jnp.sum(jnp.where(colp < NUM_VARS, entp, 0.0))
        sp1 = jnp.sum(jnp.where(colp < NUM_VARS, 0.0, entp))
        ppp_ref[...] = (jnp.exp(-sp0) + jnp.exp(-sp1)).reshape(1, 1)


@functools.partial(jax.jit, static_argnames=("interpret",))
def _run(x, W, b, vars_, interpret=False):
    b2 = b.reshape(1, NUM_TOTAL)

    q, cb, cpp, ppp = pl.pallas_call(
        _vq_kernel,
        grid=(NSTEPS,),
        in_specs=[
            pl.BlockSpec((BSZ, TT, FSZ), lambda i: (0, i, 0)),
            pl.BlockSpec((NUM_TOTAL, FSZ), lambda i: (0, 0)),
            pl.BlockSpec((1, NUM_TOTAL), lambda i: (0, 0)),
            pl.BlockSpec((1, NUM_TOTAL, VAR_DIM), lambda i: (0, 0, 0)),
        ],
        out_specs=[
            pl.BlockSpec((BSZ, TT, NUM_GROUPS * VAR_DIM), lambda i: (0, i, 0)),
            pl.BlockSpec((BSZ, TT, NUM_TOTAL), lambda i: (0, i, 0)),
            pl.BlockSpec((1, 1), lambda i: (0, 0)),
            pl.BlockSpec((1, 1), lambda i: (0, 0)),
        ],
        out_shape=[
            jax.ShapeDtypeStruct((BSZ, TSZ, NUM_GROUPS * VAR_DIM), jnp.float32),
            jax.ShapeDtypeStruct((BSZ, TSZ, NUM_TOTAL), jnp.float32),
            jax.ShapeDtypeStruct((1, 1), jnp.float32),
            jax.ShapeDtypeStruct((1, 1), jnp.float32),
        ],
        scratch_shapes=[pltpu.VMEM((8, NUM_TOTAL), jnp.float32)],
        interpret=interpret,
    )(x, W, b2, vars_)

    return q, cb.reshape(BSZ * TSZ, NUM_TOTAL), cpp[0, 0], ppp[0, 0]


def kernel(x, W, b, vars_):
    return _run(x, W, b, vars_)
